# Initial kernel scaffold; baseline (speedup 1.0000x reference)
import functools

import jax
import jax.numpy as jnp
from jax import lax
from jax.experimental import pallas as pl
from jax.experimental.pallas import tpu as pltpu
from jax.experimental.pallas import tpu_sc as plsc

NC, EC, NT, ET = 10000, 160000, 16, 128
CFEAT, CEDGE, TFEAT, TEDGE = 256, 16, 64, 16
H = 256
NOPS, OPEMB = 32, 4
NPAD = 10240
NSC, NTILE = 2, 16
ROWS_PT = NPAD // NTILE


def _sc_mesh():
    return plsc.VectorSubcoreMesh(core_axis_name="c", subcore_axis_name="s")


def _make_segsum(W):
    EPT = EC // NTILE
    NCH = EPT // 128
    REM = EPT - NCH * 128

    @functools.partial(
        pl.kernel,
        out_type=(
            jax.ShapeDtypeStruct((NPAD, W), jnp.float32),
            jax.ShapeDtypeStruct((NPAD, W), jnp.float32),
        ),
        mesh=_sc_mesh(),
        scratch_types=[
            pltpu.VMEM((128,), jnp.int32),
            pltpu.VMEM((REM,), jnp.int32),
            pltpu.VMEM((128, W), jnp.float32),
            pltpu.VMEM((REM, W), jnp.float32),
            pltpu.VMEM_SHARED((NPAD, W), jnp.float32),
        ],
    )
    def segsum(tab_a, tab_b, srcL, dstL, z, out_a, out_b,
               idx_v, idx_r, rows_v, rows_r, acc):
        c = lax.axis_index("c")
        s = lax.axis_index("s")
        r0 = s * ROWS_PT
        pltpu.sync_copy(z.at[pl.ds(r0, ROWS_PT)], acc.at[pl.ds(r0, ROWS_PT)])
        plsc.subcore_barrier()

        def run(tab):
            @pl.loop(0, NCH)
            def _(k):
                base = s * EPT + k * 128
                pltpu.sync_copy(srcL.at[pl.ds(base, 128)], idx_v)
                pltpu.sync_copy(tab.at[idx_v], rows_v)
                pltpu.sync_copy(dstL.at[pl.ds(base, 128)], idx_v)
                pltpu.sync_copy(rows_v, acc.at[idx_v], add=True)

            base = s * EPT + NCH * 128
            pltpu.sync_copy(srcL.at[pl.ds(base, REM)], idx_r)
            pltpu.sync_copy(tab.at[idx_r], rows_r)
            pltpu.sync_copy(dstL.at[pl.ds(base, REM)], idx_r)
            pltpu.sync_copy(rows_r, acc.at[idx_r], add=True)

        @pl.when(c == 0)
        def _():
            run(tab_a)

        @pl.when(c == 1)
        def _():
            run(tab_b)

        plsc.subcore_barrier()

        @pl.when(c == 0)
        def _():
            pltpu.sync_copy(acc.at[pl.ds(r0, ROWS_PT)],
                            out_a.at[pl.ds(r0, ROWS_PT)])

        @pl.when(c == 1)
        def _():
            pltpu.sync_copy(acc.at[pl.ds(r0, ROWS_PT)],
                            out_b.at[pl.ds(r0, ROWS_PT)])

    return segsum


def _make_sc_prep():
    EPC = EC // NSC
    EPT = EPC // NTILE
    NCH = EPT // 128
    REM = EPT - NCH * 128

    @functools.partial(
        pl.kernel,
        out_type=(
            jax.ShapeDtypeStruct((NSC, NPAD, 48), jnp.float32),
            jax.ShapeDtypeStruct((NSC, NPAD, 16), jnp.float32),
        ),
        mesh=_sc_mesh(),
        scratch_types=[
            pltpu.VMEM((128,), jnp.int32),
            pltpu.VMEM((REM,), jnp.int32),
            pltpu.VMEM((128, 48), jnp.float32),
            pltpu.VMEM((REM, 48), jnp.float32),
            pltpu.VMEM((128, 16), jnp.float32),
            pltpu.VMEM((REM, 16), jnp.float32),
            pltpu.VMEM_SHARED((NPAD, 48), jnp.float32),
            pltpu.VMEM_SHARED((NPAD, 16), jnp.float32),
        ],
    )
    def prep(srcL, dstL, vals48, zA, zB, onesC, outA, outB,
             idx_v, idx_r, val_v, val_r, one_v, one_r, accA, accB):
        c = lax.axis_index("c")
        s = lax.axis_index("s")
        r0 = s * ROWS_PT
        pltpu.sync_copy(zA.at[pl.ds(r0, ROWS_PT)], accA.at[pl.ds(r0, ROWS_PT)])
        pltpu.sync_copy(zB.at[pl.ds(r0, ROWS_PT)], accB.at[pl.ds(r0, ROWS_PT)])
        pltpu.sync_copy(onesC, one_v)
        pltpu.sync_copy(onesC.at[pl.ds(0, REM)], one_r)
        plsc.subcore_barrier()

        e0 = c * EPC + s * EPT

        @pl.loop(0, NCH)
        def _(k):
            base = e0 + k * 128
            pltpu.sync_copy(dstL.at[pl.ds(base, 128)], idx_v)
            pltpu.sync_copy(vals48.at[pl.ds(base, 128)], val_v)
            pltpu.sync_copy(val_v, accA.at[idx_v], add=True)
            pltpu.sync_copy(srcL.at[pl.ds(base, 128)], idx_v)
            pltpu.sync_copy(one_v, accB.at[idx_v], add=True)

        base = e0 + NCH * 128
        pltpu.sync_copy(dstL.at[pl.ds(base, REM)], idx_r)
        pltpu.sync_copy(vals48.at[pl.ds(base, REM)], val_r)
        pltpu.sync_copy(val_r, accA.at[idx_r], add=True)
        pltpu.sync_copy(srcL.at[pl.ds(base, REM)], idx_r)
        pltpu.sync_copy(one_r, accB.at[idx_r], add=True)

        plsc.subcore_barrier()

        @pl.when(c == 0)
        def _():
            pltpu.sync_copy(accA.at[pl.ds(r0, ROWS_PT)],
                            outA.at[0, pl.ds(r0, ROWS_PT)])
            pltpu.sync_copy(accB.at[pl.ds(r0, ROWS_PT)],
                            outB.at[0, pl.ds(r0, ROWS_PT)])

        @pl.when(c == 1)
        def _():
            pltpu.sync_copy(accA.at[pl.ds(r0, ROWS_PT)],
                            outA.at[1, pl.ds(r0, ROWS_PT)])
            pltpu.sync_copy(accB.at[pl.ds(r0, ROWS_PT)],
                            outB.at[1, pl.ds(r0, ROWS_PT)])

    return prep


_EBLK = 2000
_NBLK = 1000
_PBLK = 1024


def _vals48_body(ef_ref, out_ref):
    ef = ef_ref[...]
    out_ref[...] = jnp.concatenate(
        [jax.nn.sigmoid(ef), ef, jnp.ones_like(ef)], axis=1)


def _build_vals48(cedge_feats):
    return pl.pallas_call(
        _vals48_body,
        grid=(EC // _EBLK,),
        in_specs=[pl.BlockSpec((_EBLK, CEDGE), lambda i: (i, 0))],
        out_specs=pl.BlockSpec((_EBLK, 48), lambda i: (i, 0)),
        out_shape=jax.ShapeDtypeStruct((EC, 48), jnp.float32),
    )(cedge_feats)


def _degs_body(dA_ref, oB_ref, hEs_ref, hEl_ref, di_ref, do_ref):
    dA = dA_ref[0] + dA_ref[1]
    oB = oB_ref[0] + oB_ref[1]
    hEs_ref[...] = dA[:, 0:16]
    hEl_ref[...] = dA[:, 16:32]
    di_ref[...] = lax.rsqrt(jnp.maximum(dA[:, 32:33], 1.0))
    do_ref[...] = lax.rsqrt(jnp.maximum(oB[:, 0:1], 1.0))


def _build_degs(dsums, osums):
    return pl.pallas_call(
        _degs_body,
        grid=(NPAD // _PBLK,),
        in_specs=[
            pl.BlockSpec((NSC, _PBLK, 48), lambda i: (0, i, 0)),
            pl.BlockSpec((NSC, _PBLK, 16), lambda i: (0, i, 0)),
        ],
        out_specs=[
            pl.BlockSpec((_PBLK, 16), lambda i: (i, 0)),
            pl.BlockSpec((_PBLK, 16), lambda i: (i, 0)),
            pl.BlockSpec((_PBLK, 1), lambda i: (i, 0)),
            pl.BlockSpec((_PBLK, 1), lambda i: (i, 0)),
        ],
        out_shape=[
            jax.ShapeDtypeStruct((NPAD, 16), jnp.float32),
            jax.ShapeDtypeStruct((NPAD, 16), jnp.float32),
            jax.ShapeDtypeStruct((NPAD, 1), jnp.float32),
            jax.ShapeDtypeStruct((NPAD, 1), jnp.float32),
        ],
    )(dsums, osums)


def _feat0_body(cf_ref, ct_ref, emb_ref, do_ref, sa_ref, sb_ref):
    cf = cf_ref[...]
    ct = ct_ref[...]
    oh = (ct == lax.broadcasted_iota(jnp.int32, (ct.shape[0], NOPS), 1))
    emb = jnp.dot(oh.astype(jnp.float32), emb_ref[...],
                  preferred_element_type=jnp.float32)
    f0 = jnp.concatenate([cf, emb], axis=1) * do_ref[...]
    s0 = jax.nn.sigmoid(f0)
    zpad = jnp.zeros((cf.shape[0], 14), jnp.float32)
    sa_ref[...] = jnp.concatenate([s0[:, :130], zpad], axis=1)
    sb_ref[...] = jnp.concatenate([s0[:, 130:260], zpad], axis=1)


def _build_feat0(cfeats, ctypes2d, op_emb, dout_isqrt):
    return pl.pallas_call(
        _feat0_body,
        grid=(NC // _NBLK,),
        in_specs=[
            pl.BlockSpec((_NBLK, CFEAT), lambda i: (i, 0)),
            pl.BlockSpec((_NBLK, 1), lambda i: (i, 0)),
            pl.BlockSpec((NOPS, OPEMB), lambda i: (0, 0)),
            pl.BlockSpec((_NBLK, 1), lambda i: (i, 0)),
        ],
        out_specs=[
            pl.BlockSpec((_NBLK, 144), lambda i: (i, 0)),
            pl.BlockSpec((_NBLK, 144), lambda i: (i, 0)),
        ],
        out_shape=[
            jax.ShapeDtypeStruct((NC, 144), jnp.float32),
            jax.ShapeDtypeStruct((NC, 144), jnp.float32),
        ],
    )(cfeats, ctypes2d, op_emb, dout_isqrt)


def _layer_core(res, last, ha_ref, hb_ref, he_ref, wa_ref, wb_ref, we_ref,
                b_ref, di_ref, do_ref, f_ref, sa_ref, sb_ref, fo_ref):
    h = (jnp.dot(ha_ref[...], wa_ref[...], preferred_element_type=jnp.float32)
         + jnp.dot(hb_ref[...], wb_ref[...], preferred_element_type=jnp.float32)
         + jnp.dot(he_ref[...], we_ref[...], preferred_element_type=jnp.float32))
    rst = jax.nn.sigmoid(h * di_ref[...] + b_ref[...])
    x = f_ref[...] + rst if res else rst
    fn = x * do_ref[...]
    if last:
        sa_ref[...] = fn[:, :128]
        sb_ref[...] = fn[:, 128:]
    else:
        sn = jax.nn.sigmoid(fn)
        sa_ref[...] = sn[:, :128]
        sb_ref[...] = sn[:, 128:]
        fo_ref[...] = fn


def _make_layer_body(res, last):
    if last:
        def body(ha, hb, he, wa, wb, we, b, di, do, f, sa, sb):
            _layer_core(res, last, ha, hb, he, wa, wb, we, b, di, do, f,
                        sa, sb, None)
    else:
        def body(ha, hb, he, wa, wb, we, b, di, do, f, sa, sb, fo):
            _layer_core(res, last, ha, hb, he, wa, wb, we, b, di, do, f,
                        sa, sb, fo)
    return body


def _build_layer(res, last, Wa, ha, hb, hE, wa, wb, we, b2, di, do, f):
    nout = 2 if last else 3
    outs = [
        jax.ShapeDtypeStruct((NC, 128), jnp.float32),
        jax.ShapeDtypeStruct((NC, 128), jnp.float32),
        jax.ShapeDtypeStruct((NC, 256), jnp.float32),
    ][:nout]
    out_specs = [
        pl.BlockSpec((_NBLK, 128), lambda i: (i, 0)),
        pl.BlockSpec((_NBLK, 128), lambda i: (i, 0)),
        pl.BlockSpec((_NBLK, 256), lambda i: (i, 0)),
    ][:nout]
    return pl.pallas_call(
        _make_layer_body(res, last),
        grid=(NC // _NBLK,),
        in_specs=[
            pl.BlockSpec((_NBLK, Wa), lambda i: (i, 0)),
            pl.BlockSpec((_NBLK, Wa), lambda i: (i, 0)),
            pl.BlockSpec((_NBLK, 16), lambda i: (i, 0)),
            pl.BlockSpec((Wa, H), lambda i: (0, 0)),
            pl.BlockSpec((Wa, H), lambda i: (0, 0)),
            pl.BlockSpec((16, H), lambda i: (0, 0)),
            pl.BlockSpec((1, H), lambda i: (0, 0)),
            pl.BlockSpec((_NBLK, 1), lambda i: (i, 0)),
            pl.BlockSpec((_NBLK, 1), lambda i: (i, 0)),
            pl.BlockSpec((_NBLK, 256), lambda i: (i, 0)),
        ],
        out_specs=out_specs,
        out_shape=outs,
    )(ha, hb, hE, wa, wb, we, b2, di, do, f)


def _tstack_body(tf_ref, tef_ref, tsrcC_ref, tsrcR_ref, tdstR_ref,
                 tw0, tb0, tw1, tb1, tw2, tb2, tw3, tb3, out_ref):
    tws = [tw0[...], tw1[...], tw2[...], tw3[...]]
    tbs = [tb0[...], tb1[...], tb2[...], tb3[...]]
    Os = (tsrcC_ref[...] == lax.broadcasted_iota(jnp.int32, (ET, NT), 1)
          ).astype(jnp.float32)
    OsR = (lax.broadcasted_iota(jnp.int32, (NT, ET), 0) == tsrcR_ref[...]
           ).astype(jnp.float32)
    Od = (lax.broadcasted_iota(jnp.int32, (NT, ET), 0) == tdstR_ref[...]
          ).astype(jnp.float32)
    ones_e = jnp.ones((ET, 1), jnp.float32)
    tdo = lax.rsqrt(jnp.maximum(
        jnp.dot(OsR, ones_e, preferred_element_type=jnp.float32), 1.0))
    tdi = lax.rsqrt(jnp.maximum(
        jnp.dot(Od, ones_e, preferred_element_type=jnp.float32), 1.0))
    tef = tef_ref[...]
    tx = tf_ref[...]
    tres = [False, True, True, False]
    for l in range(4):
        fl = tx * tdo
        g = jnp.dot(Os, fl, preferred_element_type=jnp.float32)
        m = jnp.concatenate([g, tef], axis=1)
        if l < 3:
            m = jax.nn.sigmoid(m)
        hh = jnp.dot(Od, m, preferred_element_type=jnp.float32)
        rst = jnp.dot(hh, tws[l], preferred_element_type=jnp.float32) * tdi \
            + tbs[l]
        if l < 3:
            rst = jax.nn.sigmoid(rst)
        if tres[l]:
            rst = fl + rst
        tx = rst
    out_ref[...] = tx


def _build_tstack(tfeats, tedge_feats, tsrcC, tsrcR, tdstR, tWs, tbs):
    ins = [tfeats, tedge_feats, tsrcC, tsrcR, tdstR]
    for w, b in zip(tWs, tbs):
        ins += [w, b.reshape(1, H)]
    return pl.pallas_call(
        _tstack_body,
        out_shape=jax.ShapeDtypeStruct((NT, H), jnp.float32),
    )(*ins)


def _final_body(ha_ref, hb_ref, he_ref, wa_ref, wb_ref, we_ref, b_ref,
                di_ref, tf_ref, fwc_ref, fwt_ref, fb_ref, out_ref):
    h = (jnp.dot(ha_ref[...], wa_ref[...], preferred_element_type=jnp.float32)
         + jnp.dot(hb_ref[...], wb_ref[...], preferred_element_type=jnp.float32)
         + jnp.dot(he_ref[...], we_ref[...], preferred_element_type=jnp.float32))
    c_emb = h * di_ref[...] + b_ref[...]
    tvec = jnp.dot(tf_ref[...], fwt_ref[...], preferred_element_type=jnp.float32)
    z = (jnp.dot(c_emb, fwc_ref[...], preferred_element_type=jnp.float32)
         + tvec + fb_ref[...])
    m = jnp.max(z, axis=1, keepdims=True)
    out_ref[...] = z - m - jnp.log(
        jnp.sum(jnp.exp(z - m), axis=1, keepdims=True))


def _build_final(ha, hb, hE, wa, wb, we, b2, di, t_flat, fWc, fWt, fb2):
    return pl.pallas_call(
        _final_body,
        grid=(NC // _NBLK,),
        in_specs=[
            pl.BlockSpec((_NBLK, 128), lambda i: (i, 0)),
            pl.BlockSpec((_NBLK, 128), lambda i: (i, 0)),
            pl.BlockSpec((_NBLK, 16), lambda i: (i, 0)),
            pl.BlockSpec((128, H), lambda i: (0, 0)),
            pl.BlockSpec((128, H), lambda i: (0, 0)),
            pl.BlockSpec((16, H), lambda i: (0, 0)),
            pl.BlockSpec((1, H), lambda i: (0, 0)),
            pl.BlockSpec((_NBLK, 1), lambda i: (i, 0)),
            pl.BlockSpec((1, NT * H), lambda i: (0, 0)),
            pl.BlockSpec((H, 6), lambda i: (0, 0)),
            pl.BlockSpec((NT * H, 6), lambda i: (0, 0)),
            pl.BlockSpec((1, 6), lambda i: (0, 0)),
        ],
        out_specs=pl.BlockSpec((_NBLK, 6), lambda i: (i, 0)),
        out_shape=jax.ShapeDtypeStruct((NC, 6), jnp.float32),
    )(ha, hb, hE, wa, wb, we, b2, di, t_flat, fWc, fWt, fb2)


_segsum144 = _make_segsum(144)
_segsum128 = _make_segsum(128)
_sc_prep = _make_sc_prep()


def _pad_rows(w, rows):
    return jnp.concatenate(
        [w, jnp.zeros((rows - w.shape[0], w.shape[1]), w.dtype)], axis=0)


def kernel(cfeats, cedge_feats, ctypes, tfeats, tedge_feats, cedge_index,
           tedge_index, op_emb, cW0, cb0, cW1, cb1, cW2, cb2, cW3, cb3,
           cW4, cb4, cW5, cb5, tW0, tb0, tW1, tb1, tW2, tb2, tW3, tb3,
           fW, fb):
    f32 = jnp.float32
    src = cedge_index[0].astype(jnp.int32)
    dst = cedge_index[1].astype(jnp.int32)

    vals48 = _build_vals48(cedge_feats)
    dsums, osums = _sc_prep(src, dst, vals48,
                            jnp.zeros((NPAD, 48), f32),
                            jnp.zeros((NPAD, 16), f32),
                            jnp.ones((128, 16), f32))
    hE_sig, hE_lin, di, do = _build_degs(dsums, osums)
    hE_sig, hE_lin = hE_sig[:NC], hE_lin[:NC]
    di_n, do_n = di[:NC], do[:NC]

    s_a, s_b = _build_feat0(
        cfeats, ctypes.astype(jnp.int32).reshape(NC, 1), op_emb, do_n)

    z144 = jnp.zeros((NPAD, 144), f32)
    z128 = jnp.zeros((NPAD, 128), f32)

    cWs = [cW0, cW1, cW2, cW3, cW4, cW5]
    cbs = [cb0, cb1, cb2, cb3, cb4, cb5]
    f_prev = jnp.zeros((NC, 256), f32)
    for l in range(5):
        if l == 0:
            ha, hb = _segsum144(s_a, s_b, src, dst, z144)
            wa = _pad_rows(cW0[:130], 144)
            wb = _pad_rows(cW0[130:260], 144)
            we = cW0[260:276]
            Wa = 144
        else:
            ha, hb = _segsum128(s_a, s_b, src, dst, z128)
            wa, wb, we = cWs[l][:128], cWs[l][128:256], cWs[l][256:272]
            Wa = 128
        res = l in (1, 2, 3, 4)
        outs = _build_layer(res, l == 4, Wa, ha[:NC], hb[:NC], hE_sig,
                            wa, wb, we, cbs[l].reshape(1, H), di_n, do_n,
                            f_prev)
        if l == 4:
            s_a, s_b = outs
        else:
            s_a, s_b, f_prev = outs

    ha, hb = _segsum128(s_a, s_b, src, dst, z128)

    t_emb = _build_tstack(
        tfeats, tedge_feats,
        tedge_index[0].astype(jnp.int32).reshape(ET, 1),
        tedge_index[0].astype(jnp.int32).reshape(1, ET),
        tedge_index[1].astype(jnp.int32).reshape(1, ET),
        [tW0, tW1, tW2, tW3], [tb0, tb1, tb2, tb3])
    t_flat = t_emb.reshape(1, NT * H)

    return _build_final(ha[:NC], hb[:NC], hE_lin,
                        cW5[:128], cW5[128:256], cW5[256:272],
                        cb5.reshape(1, H), di_n, t_flat,
                        fW[:H], fW[H:], fb.reshape(1, 6))


# speedup vs baseline: 3.7117x; 3.7117x over previous
import functools

import jax
import jax.numpy as jnp
from jax import lax
from jax.experimental import pallas as pl
from jax.experimental.pallas import tpu as pltpu
from jax.experimental.pallas import tpu_sc as plsc

NC, EC, NT, ET = 10000, 160000, 16, 128
CFEAT, CEDGE, TFEAT, TEDGE = 256, 16, 64, 16
H = 256
NOPS, OPEMB = 32, 4
NPAD = 10240
NSC, NTILE = 2, 16
ROWS_PT = NPAD // NTILE


def _sc_mesh():
    return plsc.VectorSubcoreMesh(core_axis_name="c", subcore_axis_name="s")


def _make_segsum(W):
    EPT = EC // NTILE
    NCH = EPT // 128
    REM = EPT - NCH * 128

    @functools.partial(
        pl.kernel,
        out_type=(
            jax.ShapeDtypeStruct((NPAD, W), jnp.float32),
            jax.ShapeDtypeStruct((NPAD, W), jnp.float32),
        ),
        mesh=_sc_mesh(),
        scratch_types=[
            pltpu.VMEM((128,), jnp.int32),
            pltpu.VMEM((REM,), jnp.int32),
            pltpu.VMEM((128, W), jnp.float32),
            pltpu.VMEM((REM, W), jnp.float32),
            pltpu.VMEM_SHARED((NPAD, W), jnp.float32),
        ],
        compiler_params=pltpu.CompilerParams(use_tc_tiling_on_sc=False),
    )
    def segsum(tab_a, tab_b, srcL, dstL, z, out_a, out_b,
               idx_v, idx_r, rows_v, rows_r, acc):
        c = lax.axis_index("c")
        s = lax.axis_index("s")
        r0 = s * ROWS_PT
        pltpu.sync_copy(z.at[pl.ds(r0, ROWS_PT)], acc.at[pl.ds(r0, ROWS_PT)])
        plsc.subcore_barrier()

        def run(tab):
            @pl.loop(0, NCH)
            def _(k):
                base = s * EPT + k * 128
                pltpu.sync_copy(srcL.at[pl.ds(base, 128)], idx_v)
                pltpu.sync_copy(tab.at[idx_v], rows_v)
                pltpu.sync_copy(dstL.at[pl.ds(base, 128)], idx_v)
                pltpu.sync_copy(rows_v, acc.at[idx_v], add=True)

            base = s * EPT + NCH * 128
            pltpu.sync_copy(srcL.at[pl.ds(base, REM)], idx_r)
            pltpu.sync_copy(tab.at[idx_r], rows_r)
            pltpu.sync_copy(dstL.at[pl.ds(base, REM)], idx_r)
            pltpu.sync_copy(rows_r, acc.at[idx_r], add=True)

        @pl.when(c == 0)
        def _():
            run(tab_a)

        @pl.when(c == 1)
        def _():
            run(tab_b)

        plsc.subcore_barrier()

        @pl.when(c == 0)
        def _():
            pltpu.sync_copy(acc.at[pl.ds(r0, ROWS_PT)],
                            out_a.at[pl.ds(r0, ROWS_PT)])

        @pl.when(c == 1)
        def _():
            pltpu.sync_copy(acc.at[pl.ds(r0, ROWS_PT)],
                            out_b.at[pl.ds(r0, ROWS_PT)])

    return segsum


def _make_sc_prep():
    EPC = EC // NSC
    EPT = EPC // NTILE
    NCH = EPT // 128
    REM = EPT - NCH * 128

    @functools.partial(
        pl.kernel,
        out_type=(
            jax.ShapeDtypeStruct((NSC, NPAD, 48), jnp.float32),
            jax.ShapeDtypeStruct((NSC, NPAD, 16), jnp.float32),
        ),
        mesh=_sc_mesh(),
        scratch_types=[
            pltpu.VMEM((128,), jnp.int32),
            pltpu.VMEM((REM,), jnp.int32),
            pltpu.VMEM((128, 48), jnp.float32),
            pltpu.VMEM((REM, 48), jnp.float32),
            pltpu.VMEM((128, 16), jnp.float32),
            pltpu.VMEM((REM, 16), jnp.float32),
            pltpu.VMEM_SHARED((NPAD, 48), jnp.float32),
            pltpu.VMEM_SHARED((NPAD, 16), jnp.float32),
        ],
        compiler_params=pltpu.CompilerParams(use_tc_tiling_on_sc=False),
    )
    def prep(srcL, dstL, vals48, zA, zB, onesC, outA, outB,
             idx_v, idx_r, val_v, val_r, one_v, one_r, accA, accB):
        c = lax.axis_index("c")
        s = lax.axis_index("s")
        r0 = s * ROWS_PT
        pltpu.sync_copy(zA.at[pl.ds(r0, ROWS_PT)], accA.at[pl.ds(r0, ROWS_PT)])
        pltpu.sync_copy(zB.at[pl.ds(r0, ROWS_PT)], accB.at[pl.ds(r0, ROWS_PT)])
        pltpu.sync_copy(onesC, one_v)
        pltpu.sync_copy(onesC.at[pl.ds(0, REM)], one_r)
        plsc.subcore_barrier()

        e0 = c * EPC + s * EPT

        @pl.loop(0, NCH)
        def _(k):
            base = e0 + k * 128
            pltpu.sync_copy(dstL.at[pl.ds(base, 128)], idx_v)
            pltpu.sync_copy(vals48.at[pl.ds(base, 128)], val_v)
            pltpu.sync_copy(val_v, accA.at[idx_v], add=True)
            pltpu.sync_copy(srcL.at[pl.ds(base, 128)], idx_v)
            pltpu.sync_copy(one_v, accB.at[idx_v], add=True)

        base = e0 + NCH * 128
        pltpu.sync_copy(dstL.at[pl.ds(base, REM)], idx_r)
        pltpu.sync_copy(vals48.at[pl.ds(base, REM)], val_r)
        pltpu.sync_copy(val_r, accA.at[idx_r], add=True)
        pltpu.sync_copy(srcL.at[pl.ds(base, REM)], idx_r)
        pltpu.sync_copy(one_r, accB.at[idx_r], add=True)

        plsc.subcore_barrier()

        @pl.when(c == 0)
        def _():
            pltpu.sync_copy(accA.at[pl.ds(r0, ROWS_PT)],
                            outA.at[0, pl.ds(r0, ROWS_PT)])
            pltpu.sync_copy(accB.at[pl.ds(r0, ROWS_PT)],
                            outB.at[0, pl.ds(r0, ROWS_PT)])

        @pl.when(c == 1)
        def _():
            pltpu.sync_copy(accA.at[pl.ds(r0, ROWS_PT)],
                            outA.at[1, pl.ds(r0, ROWS_PT)])
            pltpu.sync_copy(accB.at[pl.ds(r0, ROWS_PT)],
                            outB.at[1, pl.ds(r0, ROWS_PT)])

    return prep


_EBLK = 2000
_NBLK = 1000
_PBLK = 1024


def _vals48_body(ef_ref, out_ref):
    ef = ef_ref[...]
    out_ref[...] = jnp.concatenate(
        [jax.nn.sigmoid(ef), ef, jnp.ones_like(ef)], axis=1)


def _build_vals48(cedge_feats):
    return pl.pallas_call(
        _vals48_body,
        grid=(EC // _EBLK,),
        in_specs=[pl.BlockSpec((_EBLK, CEDGE), lambda i: (i, 0))],
        out_specs=pl.BlockSpec((_EBLK, 48), lambda i: (i, 0)),
        out_shape=jax.ShapeDtypeStruct((EC, 48), jnp.float32),
    )(cedge_feats)


def _degs_body(dA_ref, oB_ref, hEs_ref, hEl_ref, di_ref, do_ref):
    dA = dA_ref[0] + dA_ref[1]
    oB = oB_ref[0] + oB_ref[1]
    hEs_ref[...] = dA[:, 0:16]
    hEl_ref[...] = dA[:, 16:32]
    di_ref[...] = lax.rsqrt(jnp.maximum(dA[:, 32:33], 1.0))
    do_ref[...] = lax.rsqrt(jnp.maximum(oB[:, 0:1], 1.0))


def _build_degs(dsums, osums):
    return pl.pallas_call(
        _degs_body,
        grid=(NPAD // _PBLK,),
        in_specs=[
            pl.BlockSpec((NSC, _PBLK, 48), lambda i: (0, i, 0)),
            pl.BlockSpec((NSC, _PBLK, 16), lambda i: (0, i, 0)),
        ],
        out_specs=[
            pl.BlockSpec((_PBLK, 16), lambda i: (i, 0)),
            pl.BlockSpec((_PBLK, 16), lambda i: (i, 0)),
            pl.BlockSpec((_PBLK, 1), lambda i: (i, 0)),
            pl.BlockSpec((_PBLK, 1), lambda i: (i, 0)),
        ],
        out_shape=[
            jax.ShapeDtypeStruct((NPAD, 16), jnp.float32),
            jax.ShapeDtypeStruct((NPAD, 16), jnp.float32),
            jax.ShapeDtypeStruct((NPAD, 1), jnp.float32),
            jax.ShapeDtypeStruct((NPAD, 1), jnp.float32),
        ],
    )(dsums, osums)


def _feat0_body(cf_ref, ct_ref, emb_ref, do_ref, sa_ref, sb_ref):
    cf = cf_ref[...]
    ct = ct_ref[...]
    oh = (ct == lax.broadcasted_iota(jnp.int32, (ct.shape[0], NOPS), 1))
    emb = jnp.dot(oh.astype(jnp.float32), emb_ref[...],
                  preferred_element_type=jnp.float32)
    f0 = jnp.concatenate([cf, emb], axis=1) * do_ref[...]
    s0 = jax.nn.sigmoid(f0)
    zpad = jnp.zeros((cf.shape[0], 14), jnp.float32)
    sa_ref[...] = jnp.concatenate([s0[:, :130], zpad], axis=1)
    sb_ref[...] = jnp.concatenate([s0[:, 130:260], zpad], axis=1)


def _build_feat0(cfeats, ctypes2d, op_emb, dout_isqrt):
    return pl.pallas_call(
        _feat0_body,
        grid=(NC // _NBLK,),
        in_specs=[
            pl.BlockSpec((_NBLK, CFEAT), lambda i: (i, 0)),
            pl.BlockSpec((_NBLK, 1), lambda i: (i, 0)),
            pl.BlockSpec((NOPS, OPEMB), lambda i: (0, 0)),
            pl.BlockSpec((_NBLK, 1), lambda i: (i, 0)),
        ],
        out_specs=[
            pl.BlockSpec((_NBLK, 144), lambda i: (i, 0)),
            pl.BlockSpec((_NBLK, 144), lambda i: (i, 0)),
        ],
        out_shape=[
            jax.ShapeDtypeStruct((NC, 144), jnp.float32),
            jax.ShapeDtypeStruct((NC, 144), jnp.float32),
        ],
    )(cfeats, ctypes2d, op_emb, dout_isqrt)


def _layer_core(res, last, ha_ref, hb_ref, he_ref, wa_ref, wb_ref, we_ref,
                b_ref, di_ref, do_ref, f_ref, sa_ref, sb_ref, fo_ref):
    h = (jnp.dot(ha_ref[...], wa_ref[...], preferred_element_type=jnp.float32)
         + jnp.dot(hb_ref[...], wb_ref[...], preferred_element_type=jnp.float32)
         + jnp.dot(he_ref[...], we_ref[...], preferred_element_type=jnp.float32))
    rst = jax.nn.sigmoid(h * di_ref[...] + b_ref[...])
    x = f_ref[...] + rst if res else rst
    fn = x * do_ref[...]
    if last:
        sa_ref[...] = fn[:, :128]
        sb_ref[...] = fn[:, 128:]
    else:
        sn = jax.nn.sigmoid(fn)
        sa_ref[...] = sn[:, :128]
        sb_ref[...] = sn[:, 128:]
        fo_ref[...] = fn


def _make_layer_body(res, last):
    if last:
        def body(ha, hb, he, wa, wb, we, b, di, do, f, sa, sb):
            _layer_core(res, last, ha, hb, he, wa, wb, we, b, di, do, f,
                        sa, sb, None)
    else:
        def body(ha, hb, he, wa, wb, we, b, di, do, f, sa, sb, fo):
            _layer_core(res, last, ha, hb, he, wa, wb, we, b, di, do, f,
                        sa, sb, fo)
    return body


def _build_layer(res, last, Wa, ha, hb, hE, wa, wb, we, b2, di, do, f):
    nout = 2 if last else 3
    outs = [
        jax.ShapeDtypeStruct((NC, 128), jnp.float32),
        jax.ShapeDtypeStruct((NC, 128), jnp.float32),
        jax.ShapeDtypeStruct((NC, 256), jnp.float32),
    ][:nout]
    out_specs = [
        pl.BlockSpec((_NBLK, 128), lambda i: (i, 0)),
        pl.BlockSpec((_NBLK, 128), lambda i: (i, 0)),
        pl.BlockSpec((_NBLK, 256), lambda i: (i, 0)),
    ][:nout]
    return pl.pallas_call(
        _make_layer_body(res, last),
        grid=(NC // _NBLK,),
        in_specs=[
            pl.BlockSpec((_NBLK, Wa), lambda i: (i, 0)),
            pl.BlockSpec((_NBLK, Wa), lambda i: (i, 0)),
            pl.BlockSpec((_NBLK, 16), lambda i: (i, 0)),
            pl.BlockSpec((Wa, H), lambda i: (0, 0)),
            pl.BlockSpec((Wa, H), lambda i: (0, 0)),
            pl.BlockSpec((16, H), lambda i: (0, 0)),
            pl.BlockSpec((1, H), lambda i: (0, 0)),
            pl.BlockSpec((_NBLK, 1), lambda i: (i, 0)),
            pl.BlockSpec((_NBLK, 1), lambda i: (i, 0)),
            pl.BlockSpec((_NBLK, 256), lambda i: (i, 0)),
        ],
        out_specs=out_specs,
        out_shape=outs,
    )(ha, hb, hE, wa, wb, we, b2, di, do, f)


def _tstack_body(tf_ref, tef_ref, tsrcC_ref, tsrcR_ref, tdstR_ref,
                 tw0, tb0, tw1, tb1, tw2, tb2, tw3, tb3, out_ref):
    tws = [tw0[...], tw1[...], tw2[...], tw3[...]]
    tbs = [tb0[...], tb1[...], tb2[...], tb3[...]]
    Os = (tsrcC_ref[...] == lax.broadcasted_iota(jnp.int32, (ET, NT), 1)
          ).astype(jnp.float32)
    OsR = (lax.broadcasted_iota(jnp.int32, (NT, ET), 0) == tsrcR_ref[...]
           ).astype(jnp.float32)
    Od = (lax.broadcasted_iota(jnp.int32, (NT, ET), 0) == tdstR_ref[...]
          ).astype(jnp.float32)
    ones_e = jnp.ones((ET, 1), jnp.float32)
    tdo = lax.rsqrt(jnp.maximum(
        jnp.dot(OsR, ones_e, preferred_element_type=jnp.float32), 1.0))
    tdi = lax.rsqrt(jnp.maximum(
        jnp.dot(Od, ones_e, preferred_element_type=jnp.float32), 1.0))
    tef = tef_ref[...]
    tx = tf_ref[...]
    tres = [False, True, True, False]
    for l in range(4):
        fl = tx * tdo
        g = jnp.dot(Os, fl, preferred_element_type=jnp.float32)
        m = jnp.concatenate([g, tef], axis=1)
        if l < 3:
            m = jax.nn.sigmoid(m)
        hh = jnp.dot(Od, m, preferred_element_type=jnp.float32)
        rst = jnp.dot(hh, tws[l], preferred_element_type=jnp.float32) * tdi \
            + tbs[l]
        if l < 3:
            rst = jax.nn.sigmoid(rst)
        if tres[l]:
            rst = fl + rst
        tx = rst
    out_ref[...] = tx


def _build_tstack(tfeats, tedge_feats, tsrcC, tsrcR, tdstR, tWs, tbs):
    ins = [tfeats, tedge_feats, tsrcC, tsrcR, tdstR]
    for w, b in zip(tWs, tbs):
        ins += [w, b.reshape(1, H)]
    return pl.pallas_call(
        _tstack_body,
        out_shape=jax.ShapeDtypeStruct((NT, H), jnp.float32),
    )(*ins)


def _final_body(ha_ref, hb_ref, he_ref, wa_ref, wb_ref, we_ref, b_ref,
                di_ref, tf_ref, fwc_ref, fwt_ref, fb_ref, out_ref):
    h = (jnp.dot(ha_ref[...], wa_ref[...], preferred_element_type=jnp.float32)
         + jnp.dot(hb_ref[...], wb_ref[...], preferred_element_type=jnp.float32)
         + jnp.dot(he_ref[...], we_ref[...], preferred_element_type=jnp.float32))
    c_emb = h * di_ref[...] + b_ref[...]
    tvec = jnp.dot(tf_ref[...], fwt_ref[...], preferred_element_type=jnp.float32)
    z = (jnp.dot(c_emb, fwc_ref[...], preferred_element_type=jnp.float32)
         + tvec + fb_ref[...])
    m = jnp.max(z, axis=1, keepdims=True)
    out_ref[...] = z - m - jnp.log(
        jnp.sum(jnp.exp(z - m), axis=1, keepdims=True))


def _build_final(ha, hb, hE, wa, wb, we, b2, di, t_flat, fWc, fWt, fb2):
    return pl.pallas_call(
        _final_body,
        grid=(NC // _NBLK,),
        in_specs=[
            pl.BlockSpec((_NBLK, 128), lambda i: (i, 0)),
            pl.BlockSpec((_NBLK, 128), lambda i: (i, 0)),
            pl.BlockSpec((_NBLK, 16), lambda i: (i, 0)),
            pl.BlockSpec((128, H), lambda i: (0, 0)),
            pl.BlockSpec((128, H), lambda i: (0, 0)),
            pl.BlockSpec((16, H), lambda i: (0, 0)),
            pl.BlockSpec((1, H), lambda i: (0, 0)),
            pl.BlockSpec((_NBLK, 1), lambda i: (i, 0)),
            pl.BlockSpec((1, NT * H), lambda i: (0, 0)),
            pl.BlockSpec((H, 6), lambda i: (0, 0)),
            pl.BlockSpec((NT * H, 6), lambda i: (0, 0)),
            pl.BlockSpec((1, 6), lambda i: (0, 0)),
        ],
        out_specs=pl.BlockSpec((_NBLK, 6), lambda i: (i, 0)),
        out_shape=jax.ShapeDtypeStruct((NC, 6), jnp.float32),
    )(ha, hb, hE, wa, wb, we, b2, di, t_flat, fWc, fWt, fb2)


_segsum144 = _make_segsum(144)
_segsum128 = _make_segsum(128)
_sc_prep = _make_sc_prep()


def _pad_rows(w, rows):
    return jnp.concatenate(
        [w, jnp.zeros((rows - w.shape[0], w.shape[1]), w.dtype)], axis=0)


def kernel(cfeats, cedge_feats, ctypes, tfeats, tedge_feats, cedge_index,
           tedge_index, op_emb, cW0, cb0, cW1, cb1, cW2, cb2, cW3, cb3,
           cW4, cb4, cW5, cb5, tW0, tb0, tW1, tb1, tW2, tb2, tW3, tb3,
           fW, fb):
    f32 = jnp.float32
    src = cedge_index[0].astype(jnp.int32)
    dst = cedge_index[1].astype(jnp.int32)

    vals48 = _build_vals48(cedge_feats)
    dsums, osums = _sc_prep(src, dst, vals48,
                            jnp.zeros((NPAD, 48), f32),
                            jnp.zeros((NPAD, 16), f32),
                            jnp.ones((128, 16), f32))
    hE_sig, hE_lin, di, do = _build_degs(dsums, osums)
    hE_sig, hE_lin = hE_sig[:NC], hE_lin[:NC]
    di_n, do_n = di[:NC], do[:NC]

    s_a, s_b = _build_feat0(
        cfeats, ctypes.astype(jnp.int32).reshape(NC, 1), op_emb, do_n)

    z144 = jnp.zeros((NPAD, 144), f32)
    z128 = jnp.zeros((NPAD, 128), f32)

    cWs = [cW0, cW1, cW2, cW3, cW4, cW5]
    cbs = [cb0, cb1, cb2, cb3, cb4, cb5]
    f_prev = jnp.zeros((NC, 256), f32)
    for l in range(5):
        if l == 0:
            ha, hb = _segsum144(s_a, s_b, src, dst, z144)
            wa = _pad_rows(cW0[:130], 144)
            wb = _pad_rows(cW0[130:260], 144)
            we = cW0[260:276]
            Wa = 144
        else:
            ha, hb = _segsum128(s_a, s_b, src, dst, z128)
            wa, wb, we = cWs[l][:128], cWs[l][128:256], cWs[l][256:272]
            Wa = 128
        res = l in (1, 2, 3, 4)
        outs = _build_layer(res, l == 4, Wa, ha[:NC], hb[:NC], hE_sig,
                            wa, wb, we, cbs[l].reshape(1, H), di_n, do_n,
                            f_prev)
        if l == 4:
            s_a, s_b = outs
        else:
            s_a, s_b, f_prev = outs

    ha, hb = _segsum128(s_a, s_b, src, dst, z128)

    t_emb = _build_tstack(
        tfeats, tedge_feats,
        tedge_index[0].astype(jnp.int32).reshape(ET, 1),
        tedge_index[0].astype(jnp.int32).reshape(1, ET),
        tedge_index[1].astype(jnp.int32).reshape(1, ET),
        [tW0, tW1, tW2, tW3], [tb0, tb1, tb2, tb3])
    t_flat = t_emb.reshape(1, NT * H)

    return _build_final(ha[:NC], hb[:NC], hE_lin,
                        cW5[:128], cW5[128:256], cW5[256:272],
                        cb5.reshape(1, H), di_n, t_flat,
                        fW[:H], fW[H:], fb.reshape(1, 6))
